# baseline (speedup 1.0000x reference)
import functools

import jax
import jax.numpy as jnp
from jax import lax
from jax.experimental import pallas as pl
from jax.experimental.pallas import tpu as pltpu
from jax.experimental.pallas import tpu_sc as plsc

_B = 4096
_L = 200
_DIM = 64
_V = 1000000
_P = _V // 2
_TCOLS_FULL = 7812
_CONV_ITERS = 245

_mesh = plsc.VectorSubcoreMesh(core_axis_name="c", subcore_axis_name="s")
_params = pltpu.CompilerParams(use_tc_tiling_on_sc=True, needs_layout_passes=False)


@functools.partial(
    pl.kernel,
    mesh=_mesh,
    out_type=jax.ShapeDtypeStruct((_P, 128), jnp.float32),
    scratch_types=[
        pltpu.VMEM((64, 128), jnp.float32),
        pltpu.VMEM((64, 128), jnp.float32),
        pltpu.VMEM((64, 128), jnp.float32),
        pltpu.VMEM((64, 128), jnp.float32),
        pltpu.SemaphoreType.DMA,
        pltpu.SemaphoreType.DMA,
        pltpu.SemaphoreType.DMA,
        pltpu.SemaphoreType.DMA,
    ],
    compiler_params=_params,
)
def _conv(et_hbm, tailp_hbm, t2_hbm, src0, src1, tb0, tb1,
          isem0, isem1, osem0, osem1):
    wid = lax.axis_index("s") * 2 + lax.axis_index("c")

    lanes = lax.iota(jnp.int32, 16)

    def fetch(t, buf, isem):
        @pl.when(t < _TCOLS_FULL)
        def _():
            pltpu.async_copy(
                et_hbm.at[:, pl.ds(pl.multiple_of(t * 128, 128), 128)], buf, isem)

        @pl.when(t == _TCOLS_FULL)
        def _():
            pltpu.async_copy(tailp_hbm, buf, isem)

    def wait_fetch(isem):
        pltpu.make_async_copy(
            et_hbm.at[:, pl.ds(0, 128)], src0, isem).wait()

    def transpose(t, buf, tb):
        npp = jnp.where(t < _TCOLS_FULL, 64, 32)

        def body(pp, carry):
            for k in range(8):
                rowvec = lanes + (16 * k) % 64
                colvec = jnp.full((16,), 0, jnp.int32) + 2 * pp + (k // 4)
                vals = plsc.load_gather(buf, [rowvec, colvec])
                tb[pp, pl.ds(16 * k, 16)] = vals
            return carry

        lax.fori_loop(0, npp, body, 0)

    def store(t, tb, osem):
        @pl.when(t < _TCOLS_FULL)
        def _():
            pltpu.async_copy(tb, t2_hbm.at[pl.ds(pl.multiple_of(t * 64, 64), 64)], osem)

        @pl.when(t == _TCOLS_FULL)
        def _():
            pltpu.async_copy(tb.at[pl.ds(0, 32)],
                             t2_hbm.at[pl.ds(_TCOLS_FULL * 64, 32)], osem)

    def drain_store(t, osem):
        @pl.when(t < _TCOLS_FULL)
        def _():
            pltpu.make_async_copy(tb0, t2_hbm.at[pl.ds(0, 64)], osem).wait()

        @pl.when(t == _TCOLS_FULL)
        def _():
            pltpu.make_async_copy(tb0.at[pl.ds(0, 32)],
                                  t2_hbm.at[pl.ds(0, 32)], osem).wait()

    t0 = wid
    fetch(t0, src0, isem0)

    def body(g, carry):
        tA = wid + 32 * (2 * g)
        tB = wid + 32 * (2 * g + 1)
        tN = wid + 32 * (2 * g + 2)

        @pl.when(tB <= _TCOLS_FULL)
        def _():
            fetch(tB, src1, isem1)

        @pl.when(tA <= _TCOLS_FULL)
        def _():
            wait_fetch(isem0)

            @pl.when(g > 0)
            def _():
                drain_store(tA, osem0)

            transpose(tA, src0, tb0)
            store(tA, tb0, osem0)

        @pl.when(tN <= _TCOLS_FULL)
        def _():
            fetch(tN, src0, isem0)

        @pl.when(tB <= _TCOLS_FULL)
        def _():
            wait_fetch(isem1)

            @pl.when(g > 0)
            def _():
                drain_store(tB, osem1)

            transpose(tB, src1, tb1)
            store(tB, tb1, osem1)

        return carry

    lax.fori_loop(0, (_CONV_ITERS + 1) // 2, body, 0)

    nlast = wid + 32 * (_CONV_ITERS - 1)

    def final_drain(t, osem):
        @pl.when(t <= _TCOLS_FULL)
        def _():
            drain_store(t, osem)

    final_drain(nlast, osem0)
    final_drain(wid + 32 * (_CONV_ITERS - 2), osem1)


@functools.partial(
    pl.kernel,
    mesh=_mesh,
    out_type=jax.ShapeDtypeStruct((_L, 8, 32, 8, 128), jnp.float32),
    scratch_types=[
        pltpu.VMEM((8, 128), jnp.int32),
        pltpu.VMEM((8, 128), jnp.int32),
        pltpu.VMEM((128,), jnp.int32),
        pltpu.VMEM((128,), jnp.int32),
        pltpu.VMEM((128, 128), jnp.float32),
        pltpu.VMEM((128, 128), jnp.float32),
        pltpu.VMEM((8, 8, 128), jnp.float32),
        pltpu.VMEM((8, 8, 128), jnp.float32),
        pltpu.SemaphoreType.DMA,
        pltpu.SemaphoreType.DMA,
        pltpu.SemaphoreType.DMA,
        pltpu.SemaphoreType.DMA,
        pltpu.SemaphoreType.DMA,
        pltpu.SemaphoreType.DMA,
    ],
    compiler_params=_params,
)
def _gather(sets3_hbm, t2_hbm, out_hbm, idx0, idx1, pidx0, pidx1,
            gb0, gb1, op0, op1, xsem0, xsem1, gsem0, gsem1, osem0, osem1):
    wid = lax.axis_index("s") * 2 + lax.axis_index("c")

    lanes = lax.iota(jnp.int32, 16)

    grp = pl.multiple_of((wid // 8) * 8, 8)
    row = wid % 8

    def fetch_idx(l, idxb, xsem):
        pltpu.async_copy(sets3_hbm.at[l, pl.ds(grp, 8)], idxb, xsem)

    def wait_idx(xsem):
        pltpu.make_async_copy(sets3_hbm.at[0, pl.ds(0, 8)], idx0, xsem).wait()

    def fire_gather(idxb, pidxb, gb, gsem):
        for k in range(8):
            v = idxb[row, pl.ds(16 * k, 16)]
            pidxb[pl.ds(16 * k, 16)] = lax.shift_right_logical(v, 1)
        pltpu.async_copy(t2_hbm.at[pidxb], gb, gsem)

    def wait_gather(gsem):
        pltpu.make_async_copy(t2_hbm.at[pidx0], gb0, gsem).wait()

    def transpose(idxb, gb, op):
        for k in range(8):
            v = idxb[row, pl.ds(16 * k, 16)]
            parv = lax.mul(lax.bitwise_and(v, 1), 64)
            rowvec = lanes + 16 * k

            def body(dg, carry):
                for r in range(8):
                    colvec = parv + (8 * 0 + r)
                    colvec = colvec + dg * 8
                    vals = plsc.load_gather(gb, [rowvec, colvec])
                    op[dg, r, pl.ds(16 * k, 16)] = vals
                return carry

            lax.fori_loop(0, 8, body, 0)

    def fire_out(l, op, osem):
        pltpu.async_copy(op, out_hbm.at[l, :, wid], osem)

    def drain_out(osem):
        pltpu.make_async_copy(op0, out_hbm.at[0, :, 0], osem).wait()

    fetch_idx(0, idx0, xsem0)
    fetch_idx(1, idx1, xsem1)

    def body(g, carry):
        lA = 2 * g
        lB = 2 * g + 1

        wait_idx(xsem0)
        fire_gather(idx0, pidx0, gb0, gsem0)
        wait_idx(xsem1)
        fire_gather(idx1, pidx1, gb1, gsem1)

        @pl.when(g > 0)
        def _():
            drain_out(osem0)
            drain_out(osem1)

        wait_gather(gsem0)
        transpose(idx0, gb0, op0)
        fire_out(lA, op0, osem0)

        @pl.when(g < 99)
        def _():
            fetch_idx(lA + 2, idx0, xsem0)

        wait_gather(gsem1)
        transpose(idx1, gb1, op1)
        fire_out(lB, op1, osem1)

        @pl.when(g < 99)
        def _():
            fetch_idx(lB + 2, idx1, xsem1)

        return carry

    lax.fori_loop(0, 100, body, 0)
    drain_out(osem0)
    drain_out(osem1)


def kernel(sets, E):
    s3 = sets.T.reshape(_L, 32, 128)
    et = E.T
    tailp = jnp.pad(E[_V - 64:].T, ((0, 0), (0, 64)))
    t2 = _conv(et, tailp)
    o5 = _gather(s3, t2)
    return o5.transpose(2, 4, 0, 1, 3).reshape(_B, _L, _DIM)


# speedup vs baseline: 1.0549x; 1.0549x over previous
import functools

import jax
import jax.numpy as jnp
from jax import lax
from jax.experimental import pallas as pl
from jax.experimental.pallas import tpu as pltpu
from jax.experimental.pallas import tpu_sc as plsc

_B = 4096
_L = 200
_DIM = 64
_V = 1000000
_P = _V // 2
_TCOLS_FULL = 7812
_CONV_ITERS = 245

_mesh = plsc.VectorSubcoreMesh(core_axis_name="c", subcore_axis_name="s")
_params = pltpu.CompilerParams(use_tc_tiling_on_sc=True, needs_layout_passes=False)

_NB = 4


@functools.partial(
    pl.kernel,
    mesh=_mesh,
    out_type=jax.ShapeDtypeStruct((_P, 128), jnp.float32),
    scratch_types=(
        [pltpu.VMEM((64, 128), jnp.float32) for _ in range(_NB)]
        + [pltpu.VMEM((64, 128), jnp.float32) for _ in range(_NB)]
        + [pltpu.SemaphoreType.DMA for _ in range(2 * _NB)]
    ),
    compiler_params=_params,
)
def _conv(et_hbm, tailp_hbm, t2_hbm, *refs):
    src = refs[0:_NB]
    tb = refs[_NB:2 * _NB]
    isem = refs[2 * _NB:3 * _NB]
    osem = refs[3 * _NB:4 * _NB]

    wid = lax.axis_index("s") * 2 + lax.axis_index("c")
    lanes = lax.iota(jnp.int32, 16)
    zero16 = lanes * 0

    def fetch(t, j):
        @pl.when(t < _TCOLS_FULL)
        def _():
            pltpu.async_copy(
                et_hbm.at[:, pl.ds(pl.multiple_of(t * 128, 128), 128)],
                src[j], isem[j])

        @pl.when(t == _TCOLS_FULL)
        def _():
            pltpu.async_copy(tailp_hbm, src[j], isem[j])

    def wait_fetch(j):
        pltpu.make_async_copy(
            et_hbm.at[:, pl.ds(0, 128)], src[j], isem[j]).wait()

    def transpose(t, j):
        npp = jnp.where(t < _TCOLS_FULL, 64, 32)
        buf = src[j]
        dst = tb[j]

        def body(pp, carry):
            base = zero16 + 2 * pp
            base1 = base + 1
            for k in range(8):
                rowv = lanes + (16 * k) % 64
                colv = base if k < 4 else base1
                vals = plsc.load_gather(buf, [rowv, colv])
                dst[pp, pl.ds(16 * k, 16)] = vals
            return carry

        lax.fori_loop(0, npp, body, 0)

    def store(t, j):
        @pl.when(t < _TCOLS_FULL)
        def _():
            pltpu.async_copy(
                tb[j], t2_hbm.at[pl.ds(pl.multiple_of(t * 64, 64), 64)],
                osem[j])

        @pl.when(t == _TCOLS_FULL)
        def _():
            pltpu.async_copy(tb[j].at[pl.ds(0, 32)],
                             t2_hbm.at[pl.ds(_TCOLS_FULL * 64, 32)], osem[j])

    def drain_full(j):
        pltpu.make_async_copy(tb[j], t2_hbm.at[pl.ds(0, 64)], osem[j]).wait()

    def drain_sized(t, j):
        @pl.when(t < _TCOLS_FULL)
        def _():
            drain_full(j)

        @pl.when(t == _TCOLS_FULL)
        def _():
            pltpu.make_async_copy(tb[j].at[pl.ds(0, 32)],
                                  t2_hbm.at[pl.ds(0, 32)], osem[j]).wait()

    for j in range(_NB):
        fetch(wid + 32 * j, j)

    def body(g, carry):
        for j in range(_NB):
            i = _NB * g + j
            t = wid + 32 * i

            @pl.when(t <= _TCOLS_FULL)
            def _():
                wait_fetch(j)

                @pl.when(i >= _NB)
                def _():
                    drain_full(j)

                transpose(t, j)
                store(t, j)

                tn = t + 32 * _NB

                @pl.when(tn <= _TCOLS_FULL)
                def _():
                    fetch(tn, j)

        return carry

    lax.fori_loop(0, (_CONV_ITERS + _NB - 1) // _NB, body, 0)

    for i in range(_CONV_ITERS - _NB - 1, _CONV_ITERS + 1):
        t = wid + 32 * i

        @pl.when(jnp.logical_and(t <= _TCOLS_FULL, t + 32 * _NB > _TCOLS_FULL))
        def _():
            drain_sized(t, i % _NB)


@functools.partial(
    pl.kernel,
    mesh=_mesh,
    out_type=jax.ShapeDtypeStruct((_L, 8, 32, 8, 128), jnp.float32),
    scratch_types=(
        [pltpu.VMEM((8, 128), jnp.int32) for _ in range(_NB)]
        + [pltpu.VMEM((128,), jnp.int32) for _ in range(_NB)]
        + [pltpu.VMEM((128,), jnp.int32) for _ in range(_NB)]
        + [pltpu.VMEM((128, 128), jnp.float32) for _ in range(_NB)]
        + [pltpu.VMEM((8, 8, 128), jnp.float32) for _ in range(_NB)]
        + [pltpu.SemaphoreType.DMA for _ in range(3 * _NB)]
    ),
    compiler_params=_params,
)
def _gather(sets3_hbm, t2_hbm, out_hbm, *refs):
    ix = refs[0:_NB]
    pp_ = refs[_NB:2 * _NB]
    pr = refs[2 * _NB:3 * _NB]
    gb = refs[3 * _NB:4 * _NB]
    op = refs[4 * _NB:5 * _NB]
    xsem = refs[5 * _NB:6 * _NB]
    gsem = refs[6 * _NB:7 * _NB]
    osem = refs[7 * _NB:8 * _NB]

    wid = lax.axis_index("s") * 2 + lax.axis_index("c")
    lanes = lax.iota(jnp.int32, 16)

    grp = pl.multiple_of((wid // 8) * 8, 8)
    row = wid % 8

    def fetch_idx(l, j):
        pltpu.async_copy(sets3_hbm.at[l, pl.ds(grp, 8)], ix[j], xsem[j])

    def wait_idx(j):
        pltpu.make_async_copy(
            sets3_hbm.at[0, pl.ds(0, 8)], ix[j], xsem[j]).wait()

    def fire_gather(j):
        for k in range(8):
            v = ix[j][row, pl.ds(16 * k, 16)]
            pp_[j][pl.ds(16 * k, 16)] = lax.shift_right_logical(v, 1)
            pr[j][pl.ds(16 * k, 16)] = lax.mul(lax.bitwise_and(v, 1), 64)
        pltpu.async_copy(t2_hbm.at[pp_[j]], gb[j], gsem[j])

    def wait_gather(j):
        pltpu.make_async_copy(t2_hbm.at[pp_[j]], gb[j], gsem[j]).wait()

    def transpose(j):
        buf = gb[j]
        dst = op[j]
        parvs = [pr[j][pl.ds(16 * k, 16)] for k in range(8)]
        rowvs = [lanes + 16 * k for k in range(8)]

        def dbody(d, carry):
            dg = lax.shift_right_logical(d, 3)
            r = lax.bitwise_and(d, 7)
            dsplat = jnp.full((16,), 0, jnp.int32) + d
            for k in range(8):
                colv = parvs[k] + dsplat
                vals = plsc.load_gather(buf, [rowvs[k], colv])
                dst[dg, r, pl.ds(16 * k, 16)] = vals
            return carry

        lax.fori_loop(0, 64, dbody, 0)

    def fire_out(l, j):
        pltpu.async_copy(op[j], out_hbm.at[l, :, wid], osem[j])

    def drain_out(j):
        pltpu.make_async_copy(op[j], out_hbm.at[0, :, 0], osem[j]).wait()

    for j in range(_NB):
        fetch_idx(j, j)
    for j in range(_NB):
        wait_idx(j)
        fire_gather(j)
    for j in range(_NB):
        fetch_idx(j + _NB, j)

    def body(g, carry):
        for j in range(_NB):
            l = _NB * g + j
            wait_gather(j)

            @pl.when(g > 0)
            def _():
                drain_out(j)

            transpose(j)
            fire_out(l, j)

            @pl.when(l + _NB < _L)
            def _():
                wait_idx(j)
                fire_gather(j)

            @pl.when(l + 2 * _NB < _L)
            def _():
                fetch_idx(l + 2 * _NB, j)

        return carry

    lax.fori_loop(0, _L // _NB, body, 0)
    for j in range(_NB):
        drain_out(j)


def kernel(sets, E):
    s3 = sets.T.reshape(_L, 32, 128)
    et = E.T
    tailp = jnp.pad(E[_V - 64:].T, ((0, 0), (0, 64)))
    t2 = _conv(et, tailp)
    o5 = _gather(s3, t2)
    return o5.transpose(2, 4, 0, 1, 3).reshape(_B, _L, _DIM)


# speedup vs baseline: 2.3413x; 2.2195x over previous
import functools

import jax
import jax.numpy as jnp
from jax import lax
from jax.experimental import pallas as pl
from jax.experimental.pallas import tpu as pltpu
from jax.experimental.pallas import tpu_sc as plsc

_B = 4096
_L = 200
_DIM = 64
_N = _B * _L
_NW = 32
_PER_W = _N // _NW
_CHUNK = 128
_GROUP = 4
_ROWS_G = _CHUNK * _GROUP
_NCHUNK = _PER_W // _CHUNK
_NGROUP = _PER_W // _ROWS_G

_mesh = plsc.VectorSubcoreMesh(core_axis_name="c", subcore_axis_name="s")


@functools.partial(
    pl.kernel,
    mesh=_mesh,
    out_type=jax.ShapeDtypeStruct((_N, _DIM), jnp.float32),
    scratch_types=[
        pltpu.VMEM((_NCHUNK, _CHUNK), jnp.int32),
        pltpu.VMEM((_ROWS_G, _DIM), jnp.float32),
        pltpu.VMEM((_ROWS_G, _DIM), jnp.float32),
        pltpu.SemaphoreType.DMA,
        pltpu.SemaphoreType.DMA,
    ],
    compiler_params=pltpu.CompilerParams(use_tc_tiling_on_sc=False),
)
def _gather_kernel(idx_hbm, table_hbm, out_hbm, idx_v, buf0, buf1, gsem, osem):
    wid = lax.axis_index("s") * 2 + lax.axis_index("c")
    base = wid * _PER_W

    pltpu.sync_copy(idx_hbm.at[wid], idx_v)

    def fire_gathers(g, buf):
        descs = []
        for j in range(_GROUP):
            cg = g * _GROUP + j
            descs.append(
                pltpu.async_copy(
                    table_hbm.at[idx_v.at[cg]],
                    buf.at[pl.ds(j * _CHUNK, _CHUNK)],
                    gsem,
                )
            )
        return descs

    def fire_out(g, buf):
        off = pl.multiple_of(base + g * _ROWS_G, _ROWS_G)
        return pltpu.async_copy(buf, out_hbm.at[pl.ds(off, _ROWS_G)], osem)

    def drain_out_pair():
        pltpu.make_async_copy(buf0, out_hbm.at[pl.ds(0, _ROWS_G)], osem).wait()
        pltpu.make_async_copy(buf1, out_hbm.at[pl.ds(0, _ROWS_G)], osem).wait()

    def body(g2, carry):
        g0 = g2 * 2
        g1 = g0 + 1

        @pl.when(g2 > 0)
        def _():
            drain_out_pair()

        d0 = fire_gathers(g0, buf0)
        d1 = fire_gathers(g1, buf1)
        for d in d0:
            d.wait()
        fire_out(g0, buf0)
        for d in d1:
            d.wait()
        fire_out(g1, buf1)
        return carry

    lax.fori_loop(0, _NGROUP // 2, body, 0)
    drain_out_pair()


def kernel(sets, E):
    flat = sets.reshape(_NW, _NCHUNK, _CHUNK)
    out = _gather_kernel(flat, E)
    return out.reshape(_B, _L, _DIM)
